# Initial kernel scaffold; baseline (speedup 1.0000x reference)
import jax
import jax.numpy as jnp
from jax.experimental import pallas as pl


def kernel(x, adj, W1, b1, Wc, bc, Wf, bf):
    raise NotImplementedError("write your pallas kernel here")


# speedup vs baseline: 1.3980x; 1.3980x over previous
import functools

import jax
import jax.numpy as jnp
from jax.experimental import pallas as pl
from jax.experimental.pallas import tpu as pltpu

N = 10000
NHID = 128
NEMBED = 64
NCLASS = 16
BR = 400


def _gcn_kernel(adj_ref, x_ref, W1_ref, b1_ref, Wc_ref, bc_ref, Wf_ref,
                bf_ref, c_ref, xclass_ref, fake_ref, soft_ref,
                support_ref, h_ref):
    p = pl.program_id(0)
    i = pl.program_id(1)

    @pl.when(p == 0)
    def _phase0():
        @pl.when(i == 0)
        def _init_support():
            support_ref[...] = jnp.dot(x_ref[...], W1_ref[...],
                                       preferred_element_type=jnp.float32)

        out = jnp.dot(adj_ref[...], support_ref[...],
                      preferred_element_type=jnp.float32)
        h_ref[pl.ds(i * BR, BR), :] = jnp.maximum(out + b1_ref[...], 0.0)

    @pl.when(p == 1)
    def _phase1():
        g = jnp.dot(adj_ref[...], h_ref[...],
                    preferred_element_type=jnp.float32)
        c = jnp.dot(g, Wc_ref[...],
                    preferred_element_type=jnp.float32) + bc_ref[...]
        f = jnp.dot(g, Wf_ref[...],
                    preferred_element_type=jnp.float32) + bf_ref[...]

        def log_softmax(v):
            m = jnp.max(v, axis=1, keepdims=True)
            s = jnp.sum(jnp.exp(v - m), axis=1, keepdims=True)
            return v - m - jnp.log(s)

        def elu(v):
            return jnp.where(v > 0.0, v, jnp.exp(v) - 1.0)

        c_ref[...] = c
        xclass_ref[...] = log_softmax(elu(c))
        fake_ref[...] = log_softmax(elu(f))
        m = jnp.max(c, axis=1, keepdims=True)
        e = jnp.exp(c - m)
        soft_ref[...] = e / jnp.sum(e, axis=1, keepdims=True)


@jax.jit
def kernel(x, adj, W1, b1, Wc, bc, Wf, bf):
    nsteps = N // BR
    grid = (2, nsteps)

    in_specs = [
        pl.BlockSpec((BR, N), lambda p, i: (i, 0)),
        pl.BlockSpec((N, NHID), lambda p, i: (0, 0)),
        pl.BlockSpec((NHID, NEMBED), lambda p, i: (0, 0)),
        pl.BlockSpec((1, NEMBED), lambda p, i: (0, 0)),
        pl.BlockSpec((NEMBED, NCLASS), lambda p, i: (0, 0)),
        pl.BlockSpec((1, NCLASS), lambda p, i: (0, 0)),
        pl.BlockSpec((NEMBED, 2), lambda p, i: (0, 0)),
        pl.BlockSpec((1, 2), lambda p, i: (0, 0)),
    ]
    out_specs = [
        pl.BlockSpec((BR, NCLASS), lambda p, i: (p * i, 0)),
        pl.BlockSpec((BR, NCLASS), lambda p, i: (p * i, 0)),
        pl.BlockSpec((BR, 2), lambda p, i: (p * i, 0)),
        pl.BlockSpec((BR, NCLASS), lambda p, i: (p * i, 0)),
    ]
    out_shapes = [
        jax.ShapeDtypeStruct((N, NCLASS), jnp.float32),
        jax.ShapeDtypeStruct((N, NCLASS), jnp.float32),
        jax.ShapeDtypeStruct((N, 2), jnp.float32),
        jax.ShapeDtypeStruct((N, NCLASS), jnp.float32),
    ]
    scratch_shapes = [
        pltpu.VMEM((N, NEMBED), jnp.float32),
        pltpu.VMEM((N, NEMBED), jnp.float32),
    ]

    c, x_class, x_fakereal, soft = pl.pallas_call(
        _gcn_kernel,
        grid=grid,
        in_specs=in_specs,
        out_specs=out_specs,
        out_shape=out_shapes,
        scratch_shapes=scratch_shapes,
    )(adj, x, W1, b1.reshape(1, NEMBED), Wc, bc.reshape(1, NCLASS),
      Wf, bf.reshape(1, 2))
    return (c, x_class, x_fakereal, soft)


# speedup vs baseline: 1.4007x; 1.0019x over previous
import functools

import jax
import jax.numpy as jnp
from jax.experimental import pallas as pl
from jax.experimental.pallas import tpu as pltpu

N = 10000
NHID = 128
NEMBED = 64
NCLASS = 16
BR = 400


def _gcn_kernel(adj_ref, x_ref, W1_ref, b1_ref, Wc_ref, bc_ref, Wf_ref,
                bf_ref, c_ref, xclass_ref, fake_ref, soft_ref,
                support_ref, h_ref):
    p = pl.program_id(0)
    i = pl.program_id(1)

    @pl.when(p == 0)
    def _phase0():
        @pl.when(i == 0)
        def _init_support():
            sup = jnp.dot(x_ref[...], W1_ref[...],
                          preferred_element_type=jnp.float32)
            support_ref[...] = sup.astype(jnp.bfloat16)

        out = jnp.dot(adj_ref[...].astype(jnp.bfloat16), support_ref[...],
                      preferred_element_type=jnp.float32)
        h_ref[pl.ds(i * BR, BR), :] = jnp.maximum(
            out + b1_ref[...], 0.0).astype(jnp.bfloat16)

    @pl.when(p == 1)
    def _phase1():
        g = jnp.dot(adj_ref[...].astype(jnp.bfloat16), h_ref[...],
                    preferred_element_type=jnp.float32)
        c = jnp.dot(g, Wc_ref[...],
                    preferred_element_type=jnp.float32) + bc_ref[...]
        f = jnp.dot(g, Wf_ref[...],
                    preferred_element_type=jnp.float32) + bf_ref[...]

        def log_softmax(v):
            m = jnp.max(v, axis=1, keepdims=True)
            s = jnp.sum(jnp.exp(v - m), axis=1, keepdims=True)
            return v - m - jnp.log(s)

        def elu(v):
            return jnp.where(v > 0.0, v, jnp.exp(v) - 1.0)

        c_ref[...] = c
        xclass_ref[...] = log_softmax(elu(c))
        fake_ref[...] = log_softmax(elu(f))
        m = jnp.max(c, axis=1, keepdims=True)
        e = jnp.exp(c - m)
        soft_ref[...] = e / jnp.sum(e, axis=1, keepdims=True)


@jax.jit
def kernel(x, adj, W1, b1, Wc, bc, Wf, bf):
    nsteps = N // BR
    grid = (2, nsteps)

    in_specs = [
        pl.BlockSpec((BR, N), lambda p, i: (i, 0)),
        pl.BlockSpec((N, NHID), lambda p, i: (0, 0)),
        pl.BlockSpec((NHID, NEMBED), lambda p, i: (0, 0)),
        pl.BlockSpec((1, NEMBED), lambda p, i: (0, 0)),
        pl.BlockSpec((NEMBED, NCLASS), lambda p, i: (0, 0)),
        pl.BlockSpec((1, NCLASS), lambda p, i: (0, 0)),
        pl.BlockSpec((NEMBED, 2), lambda p, i: (0, 0)),
        pl.BlockSpec((1, 2), lambda p, i: (0, 0)),
    ]
    out_specs = [
        pl.BlockSpec((BR, NCLASS), lambda p, i: (p * i, 0)),
        pl.BlockSpec((BR, NCLASS), lambda p, i: (p * i, 0)),
        pl.BlockSpec((BR, 2), lambda p, i: (p * i, 0)),
        pl.BlockSpec((BR, NCLASS), lambda p, i: (p * i, 0)),
    ]
    out_shapes = [
        jax.ShapeDtypeStruct((N, NCLASS), jnp.float32),
        jax.ShapeDtypeStruct((N, NCLASS), jnp.float32),
        jax.ShapeDtypeStruct((N, 2), jnp.float32),
        jax.ShapeDtypeStruct((N, NCLASS), jnp.float32),
    ]
    scratch_shapes = [
        pltpu.VMEM((N, NEMBED), jnp.bfloat16),
        pltpu.VMEM((N, NEMBED), jnp.bfloat16),
    ]

    c, x_class, x_fakereal, soft = pl.pallas_call(
        _gcn_kernel,
        grid=grid,
        in_specs=in_specs,
        out_specs=out_specs,
        out_shape=out_shapes,
        scratch_shapes=scratch_shapes,
    )(adj, x, W1, b1.reshape(1, NEMBED), Wc, bc.reshape(1, NCLASS),
      Wf, bf.reshape(1, 2))
    return (c, x_class, x_fakereal, soft)
